# baseline (speedup 1.0000x reference)
import functools

import jax
import jax.numpy as jnp
from jax import lax
from jax.experimental import pallas as pl
from jax.experimental.pallas import tpu as pltpu
from jax.experimental.pallas import tpu_sc as plsc

_NC = 2
_NS = 16
_NW = _NC * _NS
_CHUNK = 128


@functools.partial(jax.jit, static_argnames=("B", "D1", "D2"))
def _sc_concat_lookup(x0, x1, table1, table2, B, D1, D2):
    b_per_w = B // _NW
    nch = b_per_w // _CHUNK
    mesh = plsc.VectorSubcoreMesh(core_axis_name="c", subcore_axis_name="s")

    @functools.partial(
        pl.kernel,
        out_type=jax.ShapeDtypeStruct((B, D1 + D2), jnp.float32),
        mesh=mesh,
        scratch_types=[
            pltpu.VMEM((nch, _CHUNK), jnp.int32),
            pltpu.VMEM((nch, _CHUNK), jnp.int32),
            pltpu.VMEM((b_per_w, D1), jnp.float32),
            pltpu.VMEM((b_per_w, D2), jnp.float32),
            pltpu.SemaphoreType.DMA,
        ],
        compiler_params=pltpu.CompilerParams(use_tc_tiling_on_sc=False),
    )
    def k(x0_hbm, x1_hbm, t1_hbm, t2_hbm, out_hbm,
          idx1_v, idx2_v, rows1_v, rows2_v, sem):
        wid = lax.axis_index("s") * _NC + lax.axis_index("c")
        base = wid * b_per_w
        pltpu.sync_copy(x0_hbm.at[wid], idx1_v)
        pltpu.sync_copy(x1_hbm.at[wid], idx2_v)
        copies = []
        for j in range(nch):
            copies.append(pltpu.async_copy(
                t1_hbm.at[idx1_v.at[j]],
                rows1_v.at[pl.ds(j * _CHUNK, _CHUNK)], sem))
            copies.append(pltpu.async_copy(
                t2_hbm.at[idx2_v.at[j]],
                rows2_v.at[pl.ds(j * _CHUNK, _CHUNK)], sem))
        for c in copies:
            c.wait()
        pltpu.sync_copy(rows1_v, out_hbm.at[pl.ds(base, b_per_w), pl.ds(0, D1)])
        pltpu.sync_copy(rows2_v, out_hbm.at[pl.ds(base, b_per_w), pl.ds(D1, D2)])

    return k(x0, x1, table1, table2)


def kernel(x, table1, table2):
    B = x.shape[0]
    D1 = table1.shape[1]
    D2 = table2.shape[1]
    b_per_w = B // _NW
    nch = b_per_w // _CHUNK
    x0 = x[:, 0].astype(jnp.int32).reshape(_NW, nch, _CHUNK)
    x1 = x[:, 1].astype(jnp.int32).reshape(_NW, nch, _CHUNK)
    return _sc_concat_lookup(x0, x1, table1, table2, B, D1, D2)


# speedup vs baseline: 3.1727x; 3.1727x over previous
import functools

import jax
import jax.numpy as jnp
from jax import lax
from jax.experimental import pallas as pl
from jax.experimental.pallas import tpu as pltpu
from jax.experimental.pallas import tpu_sc as plsc

_NC = 2
_NS = 16
_NW = _NC * _NS

_V = 1000000
_D = 32
_NFULL = 7812
_VFULL = _NFULL * 128
_TAILROWS = 16
_ROWS_PER_TABLE = _NFULL * 32 + _TAILROWS
_L_ROWS = 2 * _ROWS_PER_TABLE
_NB = 245
_G = 8
_NPAIR = 16


def _make_k1():
    mesh = plsc.VectorSubcoreMesh(core_axis_name="c", subcore_axis_name="s")

    @functools.partial(
        pl.kernel,
        out_type=jax.ShapeDtypeStruct((_L_ROWS, 128), jnp.float32),
        mesh=mesh,
        scratch_types=[
            pltpu.VMEM((2, _G, _D, 128), jnp.float32),
            pltpu.VMEM((_TAILROWS, 128), jnp.float32),
            pltpu.SemaphoreType.DMA,
            pltpu.SemaphoreType.DMA,
            pltpu.SemaphoreType.DMA,
            pltpu.SemaphoreType.DMA,
        ],
    )
    def k1(t1_hbm, t2_hbm, tail1_hbm, tail2_hbm, l_hbm,
           bufs, buft, sem_r0, sem_r1, sem_w0, sem_w1):
        wid = lax.axis_index("s") * _NC + lax.axis_index("c")
        c0 = wid * _NB

        for t_hbm, tail_hbm, row_base in (
            (t1_hbm, tail1_hbm, 0),
            (t2_hbm, tail2_hbm, _ROWS_PER_TABLE),
        ):
            def blk(g, k):
                return jnp.minimum(c0 + g * _G + k, _NFULL - 1)

            def reads(g, s, sem):
                for k in range(_G):
                    c = blk(g, k)
                    pltpu.async_copy(
                        t_hbm.at[:, pl.ds(c * 128, 128)], bufs.at[s, k], sem)

            def wait_reads(s, sem):
                for k in range(_G):
                    pltpu.make_async_copy(
                        t_hbm.at[:, pl.ds(0, 128)], bufs.at[s, k], sem).wait()

            def writes(g, s, sem):
                for k in range(_G):
                    c = blk(g, k)
                    pltpu.async_copy(
                        bufs.at[s, k],
                        l_hbm.at[pl.ds(row_base + c * _D, _D)], sem)

            def wait_writes(s, sem):
                for k in range(_G):
                    pltpu.make_async_copy(
                        bufs.at[s, k], l_hbm.at[pl.ds(0, _D)], sem).wait()

            reads(0, 0, sem_r0)
            reads(1, 1, sem_r1)

            def pair(p, carry):
                for s, semr, semw in ((0, sem_r0, sem_w0), (1, sem_r1, sem_w1)):
                    g = 2 * p + s
                    wait_reads(s, semr)
                    writes(g, s, semw)
                    wait_writes(s, semw)
                    reads(g + 2, s, semr)
                return carry

            lax.fori_loop(0, _NPAIR, pair, 0)
            wait_reads(0, sem_r0)
            wait_reads(1, sem_r1)

            @pl.when(wid == 0)
            def _():
                pltpu.sync_copy(tail_hbm, buft)
                pltpu.sync_copy(
                    buft, l_hbm.at[pl.ds(row_base + _NFULL * _D, _TAILROWS)])

    return k1


def _make_k2(n_chunks):
    mesh = plsc.VectorSubcoreMesh(core_axis_name="c", subcore_axis_name="s")
    words_per_w = n_chunks * 128

    @functools.partial(
        pl.kernel,
        out_type=jax.ShapeDtypeStruct((_NW * words_per_w,), jnp.float32),
        mesh=mesh,
        scratch_types=[
            pltpu.VMEM((n_chunks, 128), jnp.int32),
            pltpu.VMEM((words_per_w,), jnp.float32),
            pltpu.SemaphoreType.DMA,
        ],
        compiler_params=pltpu.CompilerParams(use_tc_tiling_on_sc=False),
    )
    def k2(offs_hbm, lin_hbm, out_hbm, offs_v, cat_v, sem):
        wid = lax.axis_index("s") * _NC + lax.axis_index("c")
        pltpu.sync_copy(offs_hbm.at[wid], offs_v)
        for j in range(n_chunks):
            pltpu.async_copy(
                lin_hbm.at[offs_v.at[j]], cat_v.at[pl.ds(j * 128, 128)], sem)
        pltpu.make_async_copy(
            lin_hbm.at[pl.ds(0, words_per_w)], cat_v, sem).wait()
        pltpu.sync_copy(cat_v, out_hbm.at[pl.ds(wid * words_per_w, words_per_w)])

    return k2


@jax.jit
def _concat_lookup(x, table1, table2):
    B = x.shape[0]
    t1t = table1.T
    t2t = table2.T
    tail1 = t1t[:, _VFULL:].reshape(_TAILROWS, 128)
    tail2 = t2t[:, _VFULL:].reshape(_TAILROWS, 128)

    l_buf = _make_k1()(t1t, t2t, tail1, tail2)
    lin = l_buf.reshape(-1)

    d = (jnp.arange(_D, dtype=jnp.int32) * 128)[None, :]
    d_tail = (jnp.arange(_D, dtype=jnp.int32) * 64)[None, :]

    def offsets(e, table_word_base):
        e = e.astype(jnp.int32)
        full = ((e >> 7) * 4096 + (e & 127))[:, None] + d
        tail = (_NFULL * 4096 + (e - _VFULL))[:, None] + d_tail
        return table_word_base + jnp.where((e >= _VFULL)[:, None], tail, full)

    offs = jnp.concatenate(
        [offsets(x[:, 0], 0), offsets(x[:, 1], _ROWS_PER_TABLE * 128)], axis=1)
    n_chunks = (B * 64) // (_NW * 128)
    offs = offs.reshape(_NW, n_chunks, 128)

    out = _make_k2(n_chunks)(offs, lin)
    return out.reshape(B, 64)


def kernel(x, table1, table2):
    return _concat_lookup(x, table1, table2)
